# Initial kernel scaffold; baseline (speedup 1.0000x reference)
import jax
import jax.numpy as jnp
from jax.experimental import pallas as pl


def kernel(input_indices, W1, W2):
    raise NotImplementedError("write your pallas kernel here")


# speedup vs baseline: 2.6719x; 2.6719x over previous
import functools

import jax
import jax.numpy as jnp
import numpy as np
from jax.experimental import pallas as pl

_G0 = float(1.0 / np.sqrt(2.0))
_G03 = _G0 * _G0 * _G0
_BLOCK = 2048


def _composer_block(idx_ref, wbig_ref, w2big_ref, out_ref):
    idx = idx_ref[...]
    fine = idx.astype(jnp.float32) * (1.0 / 64.0)
    coarse = (idx // 2).astype(jnp.float32) * (1.0 / 32.0)
    b = idx.shape[0]
    lvl = jnp.concatenate(
        [jnp.ones((b, 1), jnp.float32), jnp.zeros((b, 1), jnp.float32)], axis=1
    )
    inp = jnp.concatenate([fine, coarse, lvl], axis=1)
    h = jnp.tanh(jnp.dot(inp, wbig_ref[...], preferred_element_type=jnp.float32))
    out_ref[...] = jnp.dot(h, w2big_ref[...], preferred_element_type=jnp.float32)


@functools.partial(jax.jit, static_argnames=())
def kernel(input_indices, W1, W2):
    n = input_indices.shape[0]
    np_rows = ((n + _BLOCK - 1) // _BLOCK) * _BLOCK
    idx = jnp.pad(input_indices, ((0, np_rows - n), (0, 0)))

    wfine = jnp.concatenate([W1[:3], jnp.zeros((5, 64), jnp.float32)], axis=0)
    wcoarse = jnp.concatenate(
        [jnp.zeros((3, 64), jnp.float32), W1[:3], W1[3:4],
         jnp.zeros((1, 64), jnp.float32)], axis=0
    )
    wbig = jnp.concatenate([wfine, wcoarse], axis=1)
    w2big = jnp.concatenate([W2, W2 * (_G0 * _G0 * _G0)], axis=0)

    grid = np_rows // _BLOCK
    out = pl.pallas_call(
        _composer_block,
        grid=(grid,),
        in_specs=[
            pl.BlockSpec((_BLOCK, 3), lambda i: (i, 0)),
            pl.BlockSpec((8, 128), lambda i: (0, 0)),
            pl.BlockSpec((128, 1), lambda i: (0, 0)),
        ],
        out_specs=pl.BlockSpec((_BLOCK, 1), lambda i: (i, 0)),
        out_shape=jax.ShapeDtypeStruct((np_rows, 1), jnp.float32),
    )(idx, wbig, w2big)
    return out[:n]


# speedup vs baseline: 4.0245x; 1.5062x over previous
import functools

import jax
import jax.numpy as jnp
import numpy as np
from jax.experimental import pallas as pl

_G0 = float(1.0 / np.sqrt(2.0))
_G03 = _G0 * _G0 * _G0
_BLOCK = 2000


def _composer_block(idx_ref, wbig_ref, w2big_ref, out_ref):
    idx = idx_ref[...]
    fine = idx.astype(jnp.float32) * (1.0 / 64.0)
    coarse = (idx // 2).astype(jnp.float32) * (1.0 / 32.0)
    b = idx.shape[0]
    lvl = jnp.concatenate(
        [jnp.ones((b, 1), jnp.float32), jnp.zeros((b, 1), jnp.float32)], axis=1
    )
    inp = jnp.concatenate([fine, coarse, lvl], axis=1)
    h = jnp.tanh(jnp.dot(inp, wbig_ref[...], preferred_element_type=jnp.float32))
    out_ref[...] = jnp.dot(h, w2big_ref[...], preferred_element_type=jnp.float32)


@functools.partial(jax.jit, static_argnames=())
def kernel(input_indices, W1, W2):
    n = input_indices.shape[0]
    np_rows = ((n + _BLOCK - 1) // _BLOCK) * _BLOCK
    idx = (input_indices if np_rows == n
           else jnp.pad(input_indices, ((0, np_rows - n), (0, 0))))

    wfine = jnp.concatenate([W1[:3], jnp.zeros((5, 64), jnp.float32)], axis=0)
    wcoarse = jnp.concatenate(
        [jnp.zeros((3, 64), jnp.float32), W1[:3], W1[3:4],
         jnp.zeros((1, 64), jnp.float32)], axis=0
    )
    wbig = jnp.concatenate([wfine, wcoarse], axis=1)
    w2big = jnp.concatenate([W2, W2 * (_G0 * _G0 * _G0)], axis=0)

    grid = np_rows // _BLOCK
    out = pl.pallas_call(
        _composer_block,
        grid=(grid,),
        in_specs=[
            pl.BlockSpec((_BLOCK, 3), lambda i: (i, 0)),
            pl.BlockSpec((8, 128), lambda i: (0, 0)),
            pl.BlockSpec((128, 1), lambda i: (0, 0)),
        ],
        out_specs=pl.BlockSpec((_BLOCK, 1), lambda i: (i, 0)),
        out_shape=jax.ShapeDtypeStruct((np_rows, 1), jnp.float32),
    )(idx, wbig, w2big)
    return out[:n]


# speedup vs baseline: 5.3852x; 1.3381x over previous
import jax
import jax.numpy as jnp
import numpy as np
from jax import lax
from jax.experimental import pallas as pl

_G0 = float(1.0 / np.sqrt(2.0))
_BLOCK = 10000


def _composer_block(idx_ref, wf_ref, wc_ref, bias_ref, w2big_ref, out_ref):
    idx = idx_ref[...]
    ff = idx.astype(jnp.float32)
    cf = lax.shift_right_arithmetic(idx, 1).astype(jnp.float32)
    pre = (
        jnp.dot(ff, wf_ref[...], preferred_element_type=jnp.float32)
        + jnp.dot(cf, wc_ref[...], preferred_element_type=jnp.float32)
        + bias_ref[...]
    )
    out_ref[...] = jnp.dot(
        jnp.tanh(pre), w2big_ref[...], preferred_element_type=jnp.float32
    )


@jax.jit
def kernel(input_indices, W1, W2):
    n = input_indices.shape[0]
    np_rows = ((n + _BLOCK - 1) // _BLOCK) * _BLOCK
    idx = (input_indices if np_rows == n
           else jnp.pad(input_indices, ((0, np_rows - n), (0, 0))))

    z = jnp.zeros((3, 64), jnp.float32)
    wf = jnp.concatenate([W1[:3] * (1.0 / 64.0), z], axis=1)
    wc = jnp.concatenate([z, W1[:3] * (1.0 / 32.0)], axis=1)
    bias = jnp.concatenate([jnp.zeros((1, 64), jnp.float32), W1[3:4]], axis=1)
    w2big = jnp.concatenate([W2, W2 * (_G0 * _G0 * _G0)], axis=0)

    grid = np_rows // _BLOCK
    out = pl.pallas_call(
        _composer_block,
        grid=(grid,),
        in_specs=[
            pl.BlockSpec((_BLOCK, 3), lambda i: (i, 0)),
            pl.BlockSpec((3, 128), lambda i: (0, 0)),
            pl.BlockSpec((3, 128), lambda i: (0, 0)),
            pl.BlockSpec((1, 128), lambda i: (0, 0)),
            pl.BlockSpec((128, 1), lambda i: (0, 0)),
        ],
        out_specs=pl.BlockSpec((_BLOCK, 1), lambda i: (i, 0)),
        out_shape=jax.ShapeDtypeStruct((np_rows, 1), jnp.float32),
    )(idx, wf, wc, bias, w2big)
    return out[:n]


# speedup vs baseline: 8.6792x; 1.6117x over previous
import jax
import jax.numpy as jnp
import numpy as np
from jax import lax
from jax.experimental import pallas as pl

_G0 = float(1.0 / np.sqrt(2.0))
_BLOCK = 12800


def _composer_block(idx_ref, wf_ref, wc_ref, bias_ref, w2row_ref, out_ref):
    idx = idx_ref[...]
    ff = idx.astype(jnp.float32)
    cf = lax.shift_right_arithmetic(idx, 1).astype(jnp.float32)
    pre = (
        jnp.dot(ff, wf_ref[...], preferred_element_type=jnp.float32)
        + jnp.dot(cf, wc_ref[...], preferred_element_type=jnp.float32)
        + bias_ref[...]
    )
    h = jnp.tanh(pre)
    out_ref[0] = lax.dot_general(
        w2row_ref[...], h, (((1,), (1,)), ((), ())),
        preferred_element_type=jnp.float32,
    )


@jax.jit
def kernel(input_indices, W1, W2):
    n = input_indices.shape[0]
    grid = (n + _BLOCK - 1) // _BLOCK
    np_rows = grid * _BLOCK

    z = jnp.zeros((3, 64), jnp.float32)
    wf = jnp.concatenate([W1[:3] * (1.0 / 64.0), z], axis=1)
    wc = jnp.concatenate([z, W1[:3] * (1.0 / 32.0)], axis=1)
    bias = jnp.concatenate([jnp.zeros((1, 64), jnp.float32), W1[3:4]], axis=1)
    w2row = jnp.concatenate([W2, W2 * (_G0 * _G0 * _G0)], axis=0).T

    out = pl.pallas_call(
        _composer_block,
        grid=(grid,),
        in_specs=[
            pl.BlockSpec((_BLOCK, 3), lambda i: (i, 0)),
            pl.BlockSpec((3, 128), lambda i: (0, 0)),
            pl.BlockSpec((3, 128), lambda i: (0, 0)),
            pl.BlockSpec((1, 128), lambda i: (0, 0)),
            pl.BlockSpec((1, 128), lambda i: (0, 0)),
        ],
        out_specs=pl.BlockSpec((1, 1, _BLOCK), lambda i: (i, 0, 0)),
        out_shape=jax.ShapeDtypeStruct((grid, 1, _BLOCK), jnp.float32),
    )(input_indices, wf, wc, bias, w2row)
    return out.reshape(np_rows, 1)[:n]
